# baseline (speedup 1.0000x reference)
import functools

import jax
import jax.numpy as jnp
from jax import lax
from jax.experimental import pallas as pl
from jax.experimental.pallas import tpu as pltpu
from jax.experimental.pallas import tpu_sc as plsc

NUM_TABLES = 26
VOCAB = 100000
EMBED_DIM = 32
BATCH = 16384
DENSE_OUT = 5
CONCAT = NUM_TABLES * EMBED_DIM

_NC = 2
_NS = 16
_NW = _NC * _NS
_HB = BATCH // 2
_L = 16
_UNROLL = 8
_GROUPS = (7, 7, 6, 6)


def _make_sc_gather(tn, t0):
    mesh = plsc.VectorSubcoreMesh(core_axis_name="c", subcore_axis_name="s")

    @functools.partial(
        pl.kernel,
        mesh=mesh,
        compiler_params=pltpu.CompilerParams(needs_layout_passes=False),
        out_type=jax.ShapeDtypeStruct((tn * EMBED_DIM * BATCH,), jnp.float32),
        scratch_types=[
            pltpu.VMEM((VOCAB,), jnp.float32),
            pltpu.VMEM((_HB,), jnp.int32),
            pltpu.VMEM((_HB,), jnp.float32),
        ],
    )
    def gather_k(idx_hbm, tab_hbm, out_hbm, row_v, idx_v, out_v):
        wid = lax.axis_index("s") * _NC + lax.axis_index("c")

        for t in range(tn):
            slab = (t * EMBED_DIM + wid) * VOCAB
            pltpu.sync_copy(tab_hbm.at[pl.ds(slab, VOCAB)], row_v)
            for h in range(2):
                pltpu.sync_copy(
                    idx_hbm.at[pl.ds((t0 + t) * BATCH + h * _HB, _HB)], idx_v
                )

                def gbody(n, cc):
                    for u in range(_UNROLL):
                        off = (n * _UNROLL + u) * _L
                        iv = idx_v[pl.ds(off, _L)]
                        out_v[pl.ds(off, _L)] = plsc.load_gather(row_v, [iv])
                    return cc

                lax.fori_loop(0, _HB // (_L * _UNROLL), gbody, 0)
                pltpu.sync_copy(
                    out_v,
                    out_hbm.at[
                        pl.ds((t * EMBED_DIM + wid) * BATCH + h * _HB, _HB)
                    ],
                )

    return gather_k


_sc_gathers = []
_t0 = 0
for _tn in _GROUPS:
    _sc_gathers.append((_make_sc_gather(_tn, _t0), _tn, _t0))
    _t0 += _tn


def _tc_dense(xs, w3, bias2d):
    bm = 2048

    def mm_k(*refs):
        x_refs = refs[:len(_GROUPS)]
        w_ref, b_ref, o_ref = refs[len(_GROUPS):]
        acc = jnp.broadcast_to(b_ref[...], (bm, DENSE_OUT))
        t0 = 0
        for g, tn in enumerate(_GROUPS):
            for t in range(tn):
                acc = acc + lax.dot_general(
                    x_refs[g][t], w_ref[t0 + t],
                    dimension_numbers=(((0,), (0,)), ((), ())),
                    preferred_element_type=jnp.float32,
                )
            t0 += tn
        o_ref[...] = acc

    in_specs = [
        pl.BlockSpec((tn, EMBED_DIM, bm), lambda i: (0, 0, i))
        for tn in _GROUPS
    ] + [
        pl.BlockSpec((NUM_TABLES, EMBED_DIM, DENSE_OUT), lambda i: (0, 0, 0)),
        pl.BlockSpec((1, DENSE_OUT), lambda i: (0, 0)),
    ]
    return pl.pallas_call(
        mm_k,
        grid=(BATCH // bm,),
        in_specs=in_specs,
        out_specs=pl.BlockSpec((bm, DENSE_OUT), lambda i: (i, 0)),
        out_shape=jax.ShapeDtypeStruct((BATCH, DENSE_OUT), jnp.float32),
    )(*xs, w3, bias2d)


def kernel(inputs, tables, W, b):
    idx1d = inputs.reshape(-1)
    tab_flat = jnp.transpose(tables, (0, 2, 1)).reshape(-1)
    xs = []
    for gk, tn, t0 in _sc_gathers:
        tab_g = lax.slice(
            tab_flat, (t0 * EMBED_DIM * VOCAB,),
            ((t0 + tn) * EMBED_DIM * VOCAB,),
        )
        x1d = gk(idx1d, tab_g)
        xs.append(x1d.reshape(tn, EMBED_DIM, BATCH))
    w3 = W.reshape(NUM_TABLES, EMBED_DIM, DENSE_OUT)
    return _tc_dense(xs, w3, b.reshape(1, DENSE_OUT))


# speedup vs baseline: 1.2071x; 1.2071x over previous
import functools

import jax
import jax.numpy as jnp
from jax import lax
from jax.experimental import pallas as pl
from jax.experimental.pallas import tpu as pltpu
from jax.experimental.pallas import tpu_sc as plsc

NUM_TABLES = 26
VOCAB = 100000
EMBED_DIM = 32
BATCH = 16384
DENSE_OUT = 5
CONCAT = NUM_TABLES * EMBED_DIM

_NC = 2
_NS = 16
_NW = _NC * _NS
_HB = BATCH // 2
_L = 16
_UNROLL = 8


def _make_sc_gather():
    mesh = plsc.VectorSubcoreMesh(core_axis_name="c", subcore_axis_name="s")

    @functools.partial(
        pl.kernel,
        mesh=mesh,
        compiler_params=pltpu.CompilerParams(needs_layout_passes=False),
        out_type=jax.ShapeDtypeStruct((NUM_TABLES * EMBED_DIM * BATCH,), jnp.float32),
        scratch_types=[
            pltpu.VMEM((VOCAB,), jnp.float32),
            pltpu.VMEM((_HB,), jnp.int32),
            pltpu.VMEM((_HB,), jnp.float32),
        ],
    )
    def gather_k(idx_hbm, tab_hbm, out_hbm, row_v, idx_v, out_v):
        wid = lax.axis_index("s") * _NC + lax.axis_index("c")

        for t in range(NUM_TABLES):
            slab = (t * EMBED_DIM + wid) * VOCAB
            pltpu.sync_copy(tab_hbm.at[pl.ds(slab, VOCAB)], row_v)
            for h in range(2):
                pltpu.sync_copy(
                    idx_hbm.at[pl.ds(t * BATCH + h * _HB, _HB)], idx_v
                )

                def gbody(n, c):
                    for u in range(_UNROLL):
                        off = (n * _UNROLL + u) * _L
                        iv = idx_v[pl.ds(off, _L)]
                        out_v[pl.ds(off, _L)] = plsc.load_gather(row_v, [iv])
                    return c

                lax.fori_loop(0, _HB // (_L * _UNROLL), gbody, 0)
                pltpu.sync_copy(
                    out_v,
                    out_hbm.at[
                        pl.ds((t * EMBED_DIM + wid) * BATCH + h * _HB, _HB)
                    ],
                )

    return gather_k


_sc_gather = _make_sc_gather()


def _tc_dense(x3, w3, bias2d):
    bm = 2048

    def mm_k(x_ref, w_ref, b_ref, o_ref):
        acc = jnp.broadcast_to(b_ref[...], (bm, DENSE_OUT))
        for t in range(NUM_TABLES):
            acc = acc + lax.dot_general(
                x_ref[t], w_ref[t],
                dimension_numbers=(((0,), (0,)), ((), ())),
                preferred_element_type=jnp.float32,
            )
        o_ref[...] = acc

    return pl.pallas_call(
        mm_k,
        grid=(BATCH // bm,),
        in_specs=[
            pl.BlockSpec((NUM_TABLES, EMBED_DIM, bm), lambda i: (0, 0, i)),
            pl.BlockSpec((NUM_TABLES, EMBED_DIM, DENSE_OUT), lambda i: (0, 0, 0)),
            pl.BlockSpec((1, DENSE_OUT), lambda i: (0, 0)),
        ],
        out_specs=pl.BlockSpec((bm, DENSE_OUT), lambda i: (i, 0)),
        out_shape=jax.ShapeDtypeStruct((BATCH, DENSE_OUT), jnp.float32),
    )(x3, w3, bias2d)


def kernel(inputs, tables, W, b):
    tabT = jnp.transpose(tables, (0, 2, 1)).reshape(-1)
    idx1d = inputs.reshape(-1)
    x1d = _sc_gather(idx1d, tabT)
    x3 = x1d.reshape(NUM_TABLES, EMBED_DIM, BATCH)
    w3 = W.reshape(NUM_TABLES, EMBED_DIM, DENSE_OUT)
    return _tc_dense(x3, w3, b.reshape(1, DENSE_OUT))
